# baseline (speedup 1.0000x reference)
import jax
import jax.numpy as jnp
from jax import lax
from jax.experimental import pallas as pl
from jax.experimental.pallas import tpu as pltpu

D = 1024
N = 2048
C = 32
TR = 256
NT = N // TR

_CH = 0
_MA = 1
_SH = 2
_R0 = 5
_OH = 37
_FW = 128


def _softmax_lanes(z):
    m = jnp.max(z, axis=-1, keepdims=True)
    e = jnp.exp(z - m)
    return e / jnp.sum(e, axis=-1, keepdims=True)


def _softplus(z):
    return jnp.maximum(z, 0.0) + jnp.log1p(jnp.exp(-jnp.abs(z)))


def _fused_kernel(x_ref,
                  chW_ref, chb_ref, maW_ref, mab_ref, shW_ref, shb_ref,
                  clW_ref, clb_ref, selW_ref, selb_ref,
                  sw_ref, sb_ref, vw_ref, vb_ref, cw_ref, cb_ref,
                  ow_ref, ob_ref, L_ref, wsc_ref,
                  out_ref,
                  wf_ref, bf_ref, cp_ref, selc_ref,
                  fR_ref, fT_ref, vS_ref, ed_ref,
                  acc_ref, m_ref, l_ref):
    i = pl.program_id(0)

    wsc = wsc_ref[...]
    sp_wc = _softplus(wsc[0:1, 0:1])
    sp_ws = _softplus(wsc[0:1, 1:2])
    neg_sp_wd = -_softplus(wsc[0:1, 2:3])
    sp_wm01 = 0.1 * _softplus(wsc[0:1, 3:4])
    eval_c = -_softplus(wsc[0:1, 4:5])

    @pl.when(i == 0)
    def _():
        wf_ref[...] = jnp.zeros((D, _FW), jnp.float32)
        wf_ref[:, 0:1] = chW_ref[...]
        wf_ref[:, 1:2] = maW_ref[...]
        wf_ref[:, 2:5] = shW_ref[...]
        wf_ref[:, 5:37] = clW_ref[...]
        wf_ref[:, 37:40] = selW_ref[0:D, :]
        bf_ref[...] = jnp.zeros((1, _FW), jnp.float32)
        bf_ref[:, 0:1] = chb_ref[...]
        bf_ref[:, 1:2] = mab_ref[...]
        bf_ref[:, 2:5] = shb_ref[...]
        bf_ref[:, 5:37] = clb_ref[...]
        bf_ref[:, 37:40] = selb_ref[...]

        L = L_ref[...]
        inv_temp = 1.0 / _softplus(wsc[0:1, 5:6])
        cp_ref[...] = jax.nn.sigmoid((L + L.T) * 0.5) * inv_temp

        meanx = jnp.mean(x_ref[...], axis=0, keepdims=True)
        ctx = jnp.dot(meanx, cw_ref[...],
                      preferred_element_type=jnp.float32) + cb_ref[...]
        selc_ref[:, 0:3] = jnp.dot(ctx, selW_ref[pl.ds(D, D), :],
                                   preferred_element_type=jnp.float32)

    r2 = lax.broadcasted_iota(jnp.int32, (TR, TR), 0)
    c2 = lax.broadcasted_iota(jnp.int32, (TR, TR), 1)
    dist = jnp.abs(i * TR + r2 - c2).astype(jnp.float32)
    ed_ref[pl.ds(i * TR, TR), :] = \
        neg_sp_wd / (1.0 + 0.1 * dist) + eval_c

    xt = x_ref[pl.ds(i * TR, TR), :]
    F = jnp.dot(xt, wf_ref[...], preferred_element_type=jnp.float32) \
        + bf_ref[...]
    charge = jnp.tanh(F[:, 0:1])
    mass = _softplus(F[:, 1:2]) + 0.5
    shell = _softmax_lanes(F[:, 2:5])
    clog = F[:, 5:37]

    iota = lax.broadcasted_iota(jnp.int32, (TR, C), 1)
    rmax = jnp.max(clog, axis=-1, keepdims=True)
    idx = jnp.min(jnp.where(clog >= rmax, iota, C), axis=-1, keepdims=True)
    oh = (iota == idx).astype(jnp.float32)
    R = jnp.dot(oh, cp_ref[...], preferred_element_type=jnp.float32)

    sel = _softmax_lanes(F[:, 37:40] + selc_ref[0:1, 0:3])
    senses = jnp.dot(xt, sw_ref[...], preferred_element_type=jnp.float32) \
        + sb_ref[...]
    x_iso = (sel[:, 0:1] * senses[:, 0:D]
             + sel[:, 1:2] * senses[:, D:2 * D]
             + sel[:, 2:3] * senses[:, 2 * D:3 * D])
    v = jnp.dot(x_iso, vw_ref[...], preferred_element_type=jnp.float32) \
        + vb_ref[...]

    feats = jnp.concatenate(
        [charge, mass, shell, R, oh, jnp.zeros((TR, _FW - _OH - C),
                                               jnp.float32)], axis=1)
    fR_ref[pl.ds(i * TR, TR), :] = feats
    fT_ref[:, pl.ds(i * TR, TR)] = feats.T
    vS_ref[pl.ds(i * TR, TR), :] = v

    ci = feats[:, _CH:_CH + 1] * sp_wc
    mi = feats[:, _MA:_MA + 1] * sp_wm01
    shi = feats[:, _SH:_SH + 3] * sp_ws
    Ri = feats[:, _R0:_R0 + C]
    tri = c2 <= r2

    acc_ref[...] = jnp.zeros((TR, D), jnp.float32)
    m_ref[...] = jnp.full((TR, 1), -1e30, jnp.float32)
    l_ref[...] = jnp.zeros((TR, 1), jnp.float32)

    def body(j, _):
        base = j * TR
        cj = fT_ref[_CH:_CH + 1, pl.ds(base, TR)]
        mj = fT_ref[_MA:_MA + 1, pl.ds(base, TR)]
        shj = fT_ref[_SH:_SH + 3, pl.ds(base, TR)] * sp_ws
        ohjT = fT_ref[_OH:_OH + C, pl.ds(base, TR)]

        gate = jnp.dot(Ri, ohjT, preferred_element_type=jnp.float32)
        shell_d = (jnp.abs(shi[:, 0:1] - shj[0:1, :])
                   + jnp.abs(shi[:, 1:2] - shj[1:2, :])
                   + jnp.abs(shi[:, 2:3] - shj[2:3, :]))
        e_d = ed_ref[pl.ds((i - j) * TR, TR), :]
        E = ci * cj + shell_d + e_d + mi * mj
        S = E * gate
        S = lax.cond(j == i,
                     lambda s: jnp.where(tri, s, -1e9),
                     lambda s: s, S)

        m_old = m_ref[...]
        m_new = jnp.maximum(m_old, jnp.max(S, axis=-1, keepdims=True))
        alpha = jnp.exp(m_old - m_new)
        p = jnp.exp(S - m_new)
        l_ref[...] = l_ref[...] * alpha + jnp.sum(p, axis=-1, keepdims=True)
        vt = vS_ref[pl.ds(base, TR), :]
        acc_ref[...] = acc_ref[...] * alpha \
            + jnp.dot(p, vt, preferred_element_type=jnp.float32)
        m_ref[...] = m_new
        return 0

    lax.fori_loop(0, i + 1, body, 0)

    o = acc_ref[...] / l_ref[...]
    out_ref[...] = jnp.dot(o, ow_ref[...],
                           preferred_element_type=jnp.float32) + ob_ref[...]


def _run(x2, p, interpret=False):
    wsc = jnp.stack([p['w_charge'], p['w_shell'], p['w_distance'],
                     p['w_mass'], p['w_valence'],
                     p['temperature']]).reshape(1, 6)

    full = lambda shape: pl.BlockSpec(shape, lambda i: (0,) * len(shape))
    out = pl.pallas_call(
        _fused_kernel,
        grid=(NT,),
        in_specs=[full((N, D)),
                  full((D, 1)), full((1, 1)), full((D, 1)), full((1, 1)),
                  full((D, 3)), full((1, 3)), full((D, C)), full((1, C)),
                  full((2 * D, 3)), full((1, 3)),
                  full((D, 3 * D)), full((1, 3 * D)), full((D, D)),
                  full((1, D)), full((D, D)), full((1, D)),
                  full((D, D)), full((1, D)), full((C, C)), full((1, 6))],
        out_specs=pl.BlockSpec((TR, D), lambda i: (i, 0)),
        out_shape=jax.ShapeDtypeStruct((N, D), jnp.float32),
        scratch_shapes=[pltpu.VMEM((D, _FW), jnp.float32),
                        pltpu.VMEM((1, _FW), jnp.float32),
                        pltpu.VMEM((C, C), jnp.float32),
                        pltpu.VMEM((1, _FW), jnp.float32),
                        pltpu.VMEM((N, _FW), jnp.float32),
                        pltpu.VMEM((_FW, N), jnp.float32),
                        pltpu.VMEM((N, D), jnp.float32),
                        pltpu.VMEM((NT * TR, TR), jnp.float32),
                        pltpu.VMEM((TR, D), jnp.float32),
                        pltpu.VMEM((TR, 1), jnp.float32),
                        pltpu.VMEM((TR, 1), jnp.float32)],
        compiler_params=pltpu.CompilerParams(
            dimension_semantics=("arbitrary",)),
        interpret=interpret,
    )(x2,
      p['charge_W'], p['charge_b'].reshape(1, 1),
      p['mass_W'], p['mass_b'].reshape(1, 1),
      p['shell_W'], p['shell_b'].reshape(1, 3),
      p['class_W'], p['class_b'].reshape(1, C),
      p['selector_W'], p['selector_b'].reshape(1, 3),
      p['sense_W'], p['sense_b'].reshape(1, 3 * D),
      p['v_W'], p['v_b'].reshape(1, D),
      p['context_W'], p['context_b'].reshape(1, D),
      p['out_W'], p['out_b'].reshape(1, D),
      p['compat_logits'], wsc)
    return out


@jax.jit
def kernel(x, params):
    b, n, d = x.shape
    out = _run(x.reshape(n, d), params)
    return out.reshape(b, n, d)


# speedup vs baseline: 1.1665x; 1.1665x over previous
import jax
import jax.numpy as jnp
from jax import lax
from jax.experimental import pallas as pl
from jax.experimental.pallas import tpu as pltpu

D = 1024
N = 2048
C = 32
TR = 256
NT = N // TR

_CH = 0
_MA = 1
_SH = 2
_R0 = 5
_OH = 37
_FW = 128


def _softmax_lanes(z):
    m = jnp.max(z, axis=-1, keepdims=True)
    e = jnp.exp(z - m)
    return e / jnp.sum(e, axis=-1, keepdims=True)


def _fused_kernel(x_ref, wf_ref, bf_ref, sw_ref, sb_ref, vw_ref, vb_ref,
                  cw_ref, cb_ref, s2_ref, cp_ref, ow_ref, ob_ref, scal_ref,
                  out_ref,
                  selc_ref, fR_ref, fT_ref, vS_ref, ed_ref,
                  acc_ref, m_ref, l_ref):
    i = pl.program_id(0)
    scal = scal_ref[...]
    sp_wc = scal[0:1, 0:1]
    sp_ws = scal[0:1, 1:2]
    neg_sp_wd = scal[0:1, 2:3]
    sp_wm01 = scal[0:1, 3:4]
    eval_c = scal[0:1, 4:5]

    @pl.when(i == 0)
    def _():
        vS_ref[...] = jnp.zeros((N, D), jnp.float32)
        meanx = jnp.mean(x_ref[...], axis=0, keepdims=True)
        ctx = jnp.dot(meanx, cw_ref[...],
                      preferred_element_type=jnp.float32) + cb_ref[...]
        selc_ref[...] = jnp.dot(ctx, s2_ref[...],
                                preferred_element_type=jnp.float32)

    r2 = lax.broadcasted_iota(jnp.int32, (TR, TR), 0)
    c2 = lax.broadcasted_iota(jnp.int32, (TR, TR), 1)
    dist = jnp.abs(i * TR + r2 - c2).astype(jnp.float32)
    ed_ref[pl.ds(i * TR, TR), :] = \
        neg_sp_wd / (1.0 + 0.1 * dist) + eval_c

    xt = x_ref[pl.ds(i * TR, TR), :]
    F = jnp.dot(xt, wf_ref[...], preferred_element_type=jnp.float32) \
        + bf_ref[...]
    charge = jnp.tanh(F[:, 0:1])
    mz = F[:, 1:2]
    mass = jnp.maximum(mz, 0.0) + jnp.log1p(jnp.exp(-jnp.abs(mz))) + 0.5
    shell = _softmax_lanes(F[:, 2:5])
    clog = F[:, 5:37]

    iota = lax.broadcasted_iota(jnp.int32, (TR, C), 1)
    rmax = jnp.max(clog, axis=-1, keepdims=True)
    idx = jnp.min(jnp.where(clog >= rmax, iota, C), axis=-1, keepdims=True)
    oh = (iota == idx).astype(jnp.float32)
    R = jnp.dot(oh, cp_ref[...], preferred_element_type=jnp.float32)[:, :C]

    sel = _softmax_lanes(F[:, 37:40] + selc_ref[0:1, 0:3])
    senses = jnp.dot(xt, sw_ref[...], preferred_element_type=jnp.float32) \
        + sb_ref[...]
    x_iso = (sel[:, 0:1] * senses[:, 0:D]
             + sel[:, 1:2] * senses[:, D:2 * D]
             + sel[:, 2:3] * senses[:, 2 * D:3 * D])
    v = jnp.dot(x_iso, vw_ref[...], preferred_element_type=jnp.float32) \
        + vb_ref[...]

    feats = jnp.concatenate(
        [charge, mass, shell, R, oh, jnp.zeros((TR, _FW - _OH - C),
                                               jnp.float32)], axis=1)
    fR_ref[pl.ds(i * TR, TR), :] = feats
    fT_ref[:, pl.ds(i * TR, TR)] = feats.T
    vS_ref[pl.ds(i * TR, TR), :] = v

    ci = feats[:, _CH:_CH + 1] * sp_wc
    mi = feats[:, _MA:_MA + 1] * sp_wm01
    shi = feats[:, _SH:_SH + 3] * sp_ws
    Ri = feats[:, _R0:_R0 + C]
    CW = 2 * TR
    rowg = i * TR + lax.broadcasted_iota(jnp.int32, (TR, CW), 0)
    colrel = lax.broadcasted_iota(jnp.int32, (TR, CW), 1)

    acc_ref[...] = jnp.zeros((TR, D), jnp.float32)
    m_ref[...] = jnp.full((TR, 1), -1e30, jnp.float32)
    l_ref[...] = jnp.zeros((TR, 1), jnp.float32)

    def body(k, _):
        base = k * CW
        cj = fT_ref[_CH:_CH + 1, pl.ds(base, CW)]
        mj = fT_ref[_MA:_MA + 1, pl.ds(base, CW)]
        shj = fT_ref[_SH:_SH + 3, pl.ds(base, CW)] * sp_ws
        ohjT = fT_ref[_OH:_OH + C, pl.ds(base, CW)]

        gate = jnp.dot(Ri, ohjT, preferred_element_type=jnp.float32)
        shell_d = (jnp.abs(shi[:, 0:1] - shj[0:1, :])
                   + jnp.abs(shi[:, 1:2] - shj[1:2, :])
                   + jnp.abs(shi[:, 2:3] - shj[2:3, :]))
        d1 = i - 2 * k
        d0 = jnp.maximum(d1 - 1, 0)
        e_d = jnp.concatenate([ed_ref[pl.ds(d1 * TR, TR), :],
                               ed_ref[pl.ds(d0 * TR, TR), :]], axis=1)
        E = ci * cj + shell_d + e_d + mi * mj
        S = E * gate
        S = jnp.where(base + colrel <= rowg, S, -1e9)

        m_old = m_ref[...]
        m_new = jnp.maximum(m_old, jnp.max(S, axis=-1, keepdims=True))
        alpha = jnp.exp(m_old - m_new)
        p = jnp.exp(S - m_new)
        l_ref[...] = l_ref[...] * alpha + jnp.sum(p, axis=-1, keepdims=True)
        vt = vS_ref[pl.ds(base, CW), :]
        acc_ref[...] = acc_ref[...] * alpha \
            + jnp.dot(p, vt, preferred_element_type=jnp.float32)
        m_ref[...] = m_new
        return 0

    lax.fori_loop(0, i // 2 + 1, body, 0)

    o = acc_ref[...] / l_ref[...]
    out_ref[...] = jnp.dot(o, ow_ref[...],
                           preferred_element_type=jnp.float32) + ob_ref[...]


def _run(x2, p, interpret=False):
    sp = jax.nn.softplus
    W_feat = jnp.concatenate(
        [p['charge_W'], p['mass_W'], p['shell_W'], p['class_W'],
         p['selector_W'][:D]], axis=1)
    W_feat = jnp.pad(W_feat, ((0, 0), (0, _FW - W_feat.shape[1])))
    b_feat = jnp.concatenate(
        [p['charge_b'], p['mass_b'], p['shell_b'], p['class_b'],
         p['selector_b']])
    b_feat = jnp.pad(b_feat, (0, _FW - b_feat.shape[0])).reshape(1, _FW)
    L = p['compat_logits']
    inv_temp = 1.0 / sp(p['temperature'])
    compat = (jax.nn.sigmoid((L + L.T) / 2.0) * inv_temp)
    compat = jnp.pad(compat, ((0, 0), (0, _FW - C)))
    selW2 = jnp.pad(p['selector_W'][D:], ((0, 0), (0, _FW - 3)))
    ctx_b = p['context_b'].reshape(1, D)
    v_b = p['v_b'].reshape(1, D)
    out_b = p['out_b'].reshape(1, D)
    scal = jnp.stack([sp(p['w_charge']), sp(p['w_shell']),
                      -sp(p['w_distance']), 0.1 * sp(p['w_mass']),
                      -sp(p['w_valence'])])
    scal = jnp.pad(scal, (0, _FW - 5)).reshape(1, _FW)

    full = lambda shape: pl.BlockSpec(shape, lambda i: (0,) * len(shape))
    out = pl.pallas_call(
        _fused_kernel,
        grid=(NT,),
        in_specs=[full((N, D)), full((D, _FW)), full((1, _FW)),
                  full((D, 3 * D)), full((1, 3 * D)), full((D, D)),
                  full((1, D)), full((D, D)), full((1, D)), full((D, _FW)),
                  full((C, _FW)), full((D, D)), full((1, D)),
                  full((1, _FW))],
        out_specs=pl.BlockSpec((TR, D), lambda i: (i, 0)),
        out_shape=jax.ShapeDtypeStruct((N, D), jnp.float32),
        scratch_shapes=[pltpu.VMEM((1, _FW), jnp.float32),
                        pltpu.VMEM((N, _FW), jnp.float32),
                        pltpu.VMEM((_FW, N), jnp.float32),
                        pltpu.VMEM((N, D), jnp.float32),
                        pltpu.VMEM((NT * TR, TR), jnp.float32),
                        pltpu.VMEM((TR, D), jnp.float32),
                        pltpu.VMEM((TR, 1), jnp.float32),
                        pltpu.VMEM((TR, 1), jnp.float32)],
        compiler_params=pltpu.CompilerParams(
            dimension_semantics=("arbitrary",)),
        interpret=interpret,
    )(x2, W_feat, b_feat, p['sense_W'], p['sense_b'].reshape(1, 3 * D),
      p['v_W'], v_b, p['context_W'], ctx_b, selW2, compat,
      p['out_W'], out_b, scal)
    return out


@jax.jit
def kernel(x, params):
    b, n, d = x.shape
    out = _run(x.reshape(n, d), params)
    return out.reshape(b, n, d)


# speedup vs baseline: 1.2049x; 1.0329x over previous
import jax
import jax.numpy as jnp
from jax import lax
from jax.experimental import pallas as pl
from jax.experimental.pallas import tpu as pltpu

D = 1024
N = 2048
C = 32
TR = 256
NT = N // TR

_CH = 0
_MA = 1
_SH = 2
_R0 = 5
_OH = 37
_FW = 128


def _softmax_lanes(z):
    m = jnp.max(z, axis=-1, keepdims=True)
    e = jnp.exp(z - m)
    return e / jnp.sum(e, axis=-1, keepdims=True)


def _fused_kernel(x_ref, wf_ref, bf_ref, sw_ref, sb_ref, vw_ref, vb_ref,
                  cw_ref, cb_ref, s2_ref, cp_ref, ow_ref, ob_ref, scal_ref,
                  out_ref,
                  selc_ref, fR_ref, fT_ref, vS_ref, ed_ref,
                  acc_ref, m_ref, l_ref):
    i = pl.program_id(0)
    scal = scal_ref[...]
    sp_wc = scal[0:1, 0:1]
    sp_ws = scal[0:1, 1:2]
    neg_sp_wd = scal[0:1, 2:3]
    sp_wm01 = scal[0:1, 3:4]
    eval_c = scal[0:1, 4:5]

    @pl.when(i == 0)
    def _():
        vS_ref[...] = jnp.zeros((N, D), jnp.float32)
        meanx = jnp.mean(x_ref[...], axis=0, keepdims=True)
        ctx = jnp.dot(meanx, cw_ref[...],
                      preferred_element_type=jnp.float32) + cb_ref[...]
        selc_ref[...] = jnp.dot(ctx, s2_ref[...],
                                preferred_element_type=jnp.float32)

    r2 = lax.broadcasted_iota(jnp.int32, (TR, TR), 0)
    c2 = lax.broadcasted_iota(jnp.int32, (TR, TR), 1)
    dist = jnp.abs(i * TR + r2 - c2).astype(jnp.float32)
    ed_ref[pl.ds(i * TR, TR), :] = \
        neg_sp_wd / (1.0 + 0.1 * dist) + eval_c

    xt = x_ref[pl.ds(i * TR, TR), :]
    F = jnp.dot(xt, wf_ref[...], preferred_element_type=jnp.float32) \
        + bf_ref[...]
    charge = jnp.tanh(F[:, 0:1])
    mz = F[:, 1:2]
    mass = jnp.maximum(mz, 0.0) + jnp.log1p(jnp.exp(-jnp.abs(mz))) + 0.5
    shell = _softmax_lanes(F[:, 2:5])
    clog = F[:, 5:37]

    iota = lax.broadcasted_iota(jnp.int32, (TR, C), 1)
    rmax = jnp.max(clog, axis=-1, keepdims=True)
    idx = jnp.min(jnp.where(clog >= rmax, iota, C), axis=-1, keepdims=True)
    oh = (iota == idx).astype(jnp.float32)
    R = jnp.dot(oh, cp_ref[...], preferred_element_type=jnp.float32)[:, :C]

    sel = _softmax_lanes(F[:, 37:40] + selc_ref[0:1, 0:3])
    senses = jnp.dot(xt, sw_ref[...], preferred_element_type=jnp.float32) \
        + sb_ref[...]
    x_iso = (sel[:, 0:1] * senses[:, 0:D]
             + sel[:, 1:2] * senses[:, D:2 * D]
             + sel[:, 2:3] * senses[:, 2 * D:3 * D])
    v = jnp.dot(x_iso, vw_ref[...], preferred_element_type=jnp.float32) \
        + vb_ref[...]

    feats = jnp.concatenate(
        [charge, mass, shell, R, oh, jnp.zeros((TR, _FW - _OH - C),
                                               jnp.float32)], axis=1)
    fR_ref[pl.ds(i * TR, TR), :] = feats
    fT_ref[:, pl.ds(i * TR, TR)] = feats.T
    vS_ref[pl.ds(i * TR, TR), :] = v

    ci = feats[:, _CH:_CH + 1] * sp_wc
    mi = feats[:, _MA:_MA + 1] * sp_wm01
    shi = feats[:, _SH:_SH + 3] * sp_ws
    Ri = feats[:, _R0:_R0 + C]
    CW = 4 * TR
    rowg = i * TR + lax.broadcasted_iota(jnp.int32, (TR, CW), 0)
    colrel = lax.broadcasted_iota(jnp.int32, (TR, CW), 1)

    acc_ref[...] = jnp.zeros((TR, D), jnp.float32)
    m_ref[...] = jnp.full((TR, 1), -1e30, jnp.float32)
    l_ref[...] = jnp.zeros((TR, 1), jnp.float32)

    def body(k, _):
        base = k * CW
        cj = fT_ref[_CH:_CH + 1, pl.ds(base, CW)]
        mj = fT_ref[_MA:_MA + 1, pl.ds(base, CW)]
        shj = fT_ref[_SH:_SH + 3, pl.ds(base, CW)] * sp_ws
        ohjT = fT_ref[_OH:_OH + C, pl.ds(base, CW)]

        gate = jnp.dot(Ri, ohjT, preferred_element_type=jnp.float32)
        shell_d = (jnp.abs(shi[:, 0:1] - shj[0:1, :])
                   + jnp.abs(shi[:, 1:2] - shj[1:2, :])
                   + jnp.abs(shi[:, 2:3] - shj[2:3, :]))
        d1 = i - 4 * k
        dd = lambda t: jnp.maximum(d1 - t, 0) * TR
        e_d = jnp.concatenate([ed_ref[pl.ds(dd(0), TR), :],
                               ed_ref[pl.ds(dd(1), TR), :],
                               ed_ref[pl.ds(dd(2), TR), :],
                               ed_ref[pl.ds(dd(3), TR), :]], axis=1)
        E = ci * cj + shell_d + e_d + mi * mj
        S = E * gate
        S = jnp.where(base + colrel <= rowg, S, -1e9)

        m_old = m_ref[...]
        m_new = jnp.maximum(m_old, jnp.max(S, axis=-1, keepdims=True))
        alpha = jnp.exp(m_old - m_new)
        p = jnp.exp(S - m_new)
        l_ref[...] = l_ref[...] * alpha + jnp.sum(p, axis=-1, keepdims=True)
        vt = vS_ref[pl.ds(base, CW), :]
        acc_ref[...] = acc_ref[...] * alpha \
            + jnp.dot(p, vt, preferred_element_type=jnp.float32)
        m_ref[...] = m_new
        return 0

    lax.fori_loop(0, i // 4 + 1, body, 0)

    o = acc_ref[...] / l_ref[...]
    out_ref[...] = jnp.dot(o, ow_ref[...],
                           preferred_element_type=jnp.float32) + ob_ref[...]


def _run(x2, p, interpret=False):
    sp = jax.nn.softplus
    W_feat = jnp.concatenate(
        [p['charge_W'], p['mass_W'], p['shell_W'], p['class_W'],
         p['selector_W'][:D]], axis=1)
    W_feat = jnp.pad(W_feat, ((0, 0), (0, _FW - W_feat.shape[1])))
    b_feat = jnp.concatenate(
        [p['charge_b'], p['mass_b'], p['shell_b'], p['class_b'],
         p['selector_b']])
    b_feat = jnp.pad(b_feat, (0, _FW - b_feat.shape[0])).reshape(1, _FW)
    L = p['compat_logits']
    inv_temp = 1.0 / sp(p['temperature'])
    compat = (jax.nn.sigmoid((L + L.T) / 2.0) * inv_temp)
    compat = jnp.pad(compat, ((0, 0), (0, _FW - C)))
    selW2 = jnp.pad(p['selector_W'][D:], ((0, 0), (0, _FW - 3)))
    ctx_b = p['context_b'].reshape(1, D)
    v_b = p['v_b'].reshape(1, D)
    out_b = p['out_b'].reshape(1, D)
    scal = jnp.stack([sp(p['w_charge']), sp(p['w_shell']),
                      -sp(p['w_distance']), 0.1 * sp(p['w_mass']),
                      -sp(p['w_valence'])])
    scal = jnp.pad(scal, (0, _FW - 5)).reshape(1, _FW)

    full = lambda shape: pl.BlockSpec(shape, lambda i: (0,) * len(shape))
    out = pl.pallas_call(
        _fused_kernel,
        grid=(NT,),
        in_specs=[full((N, D)), full((D, _FW)), full((1, _FW)),
                  full((D, 3 * D)), full((1, 3 * D)), full((D, D)),
                  full((1, D)), full((D, D)), full((1, D)), full((D, _FW)),
                  full((C, _FW)), full((D, D)), full((1, D)),
                  full((1, _FW))],
        out_specs=pl.BlockSpec((TR, D), lambda i: (i, 0)),
        out_shape=jax.ShapeDtypeStruct((N, D), jnp.float32),
        scratch_shapes=[pltpu.VMEM((1, _FW), jnp.float32),
                        pltpu.VMEM((N, _FW), jnp.float32),
                        pltpu.VMEM((_FW, N), jnp.float32),
                        pltpu.VMEM((N, D), jnp.float32),
                        pltpu.VMEM((NT * TR, TR), jnp.float32),
                        pltpu.VMEM((TR, D), jnp.float32),
                        pltpu.VMEM((TR, 1), jnp.float32),
                        pltpu.VMEM((TR, 1), jnp.float32)],
        compiler_params=pltpu.CompilerParams(
            dimension_semantics=("arbitrary",)),
        interpret=interpret,
    )(x2, W_feat, b_feat, p['sense_W'], p['sense_b'].reshape(1, 3 * D),
      p['v_W'], v_b, p['context_W'], ctx_b, selW2, compat,
      p['out_W'], out_b, scal)
    return out


@jax.jit
def kernel(x, params):
    b, n, d = x.shape
    out = _run(x.reshape(n, d), params)
    return out.reshape(b, n, d)
